# baseline (speedup 1.0000x reference)
import jax
import jax.numpy as jnp
from jax.experimental import pallas as pl
from jax.experimental.pallas import tpu as pltpu

B, S, DIM = 8, 576, 256
K = 1024
NQ = 8
N = B * S
COMMIT_W = 0.25

ROW_BLK = 1152
N_BLKS = N // ROW_BLK


def _rvq_body(x_ref, cb_ref, cbw_ref, cbsq_ref, q_ref, *out_refs):
    idx_refs = out_refs[:NQ]
    counts_ref = out_refs[NQ]
    commit_ref = out_refs[NQ + 1]

    x = x_ref[...]
    iota = jax.lax.broadcasted_iota(jnp.int32, (ROW_BLK, K), 1)
    r = x
    qsum = jnp.zeros_like(x)
    rsq = jnp.sum(r ** 2, axis=1, keepdims=True)
    for level in range(NQ):
        cb = cb_ref[level * K:(level + 1) * K, :]
        cbsq = cbsq_ref[level:level + 1, :]
        sc = jax.lax.dot_general(
            r, cb, (((1,), (1,)), ((), ())),
            preferred_element_type=jnp.float32)
        d = (rsq - 2.0 * sc) + cbsq
        idx = jnp.argmin(d, axis=1).astype(jnp.int32)
        idx_refs[level][...] = idx[None, None, :]
        ohi = (iota == idx[:, None]).astype(jnp.int8)
        dn = (((1,), (0,)), ((), ()))
        sl = slice(level * K, (level + 1) * K)
        g = jax.lax.dot_general(ohi, cbw_ref[sl, :], dn,
                                preferred_element_type=jnp.int32)
        bits = (g[:, 0 * DIM:1 * DIM] + 128).astype(jnp.uint32)
        for c in range(1, 4):
            bits = bits | ((g[:, c * DIM:(c + 1) * DIM] + 128)
                           .astype(jnp.uint32) << (8 * c))
        q = jax.lax.bitcast_convert_type(bits, jnp.float32)
        counts_ref[0, level:level + 1, :] = jax.lax.dot_general(
            jnp.ones((1, ROW_BLK), jnp.int8), ohi,
            (((1,), (0,)), ((), ())),
            preferred_element_type=jnp.int32).astype(jnp.float32)
        qsum = qsum + q
        r = r - q
        rsq = jnp.sum(r ** 2, axis=1, keepdims=True)
        commit_ref[0, level:level + 1, :] = jnp.full((1, 128), jnp.sum(rsq))
    q_ref[...] = x + (qsum - x)


def _finalize_body(counts_ref, commit_ref, counts_out_ref, com_ref, perp_ref):
    counts = jnp.sum(counts_ref[...], axis=0)
    counts_out_ref[...] = counts
    p = counts / N
    ent = jnp.sum(p * jnp.log(p + 1e-10), axis=1, keepdims=True)
    perps = jnp.exp(-ent)
    perp_ref[...] = jnp.full((8, 128), jnp.sum(perps) / NQ)
    commit = jnp.sum(commit_ref[...][:, :, 0:1]) / (N * DIM)
    com_ref[...] = jnp.full((8, 128), commit * COMMIT_W)


def kernel(x, codebooks):
    x_flat = x.reshape(N, DIM)
    cb_flat = codebooks.reshape(NQ * K, DIM)
    cb_bits = jax.lax.bitcast_convert_type(cb_flat, jnp.uint32)
    cbw = jnp.concatenate(
        [((cb_bits >> (8 * c)) & 0xFF).astype(jnp.int32) - 128
         for c in range(4)],
        axis=1).astype(jnp.int8)
    cbsq_all = jnp.sum(codebooks ** 2, axis=-1)

    outs = pl.pallas_call(
        _rvq_body,
        grid=(N_BLKS,),
        in_specs=[
            pl.BlockSpec((ROW_BLK, DIM), lambda i: (i, 0)),
            pl.BlockSpec((NQ * K, DIM), lambda i: (0, 0)),
            pl.BlockSpec((NQ * K, 4 * DIM), lambda i: (0, 0)),
            pl.BlockSpec((NQ, K), lambda i: (0, 0)),
        ],
        out_specs=[pl.BlockSpec((ROW_BLK, DIM), lambda i: (i, 0))]
        + [pl.BlockSpec((1, 1, ROW_BLK), lambda i: (i, 0, 0))
           for _ in range(NQ)]
        + [
            pl.BlockSpec((1, NQ, K), lambda i: (i, 0, 0)),
            pl.BlockSpec((1, NQ, 128), lambda i: (i, 0, 0)),
        ],
        out_shape=[jax.ShapeDtypeStruct((N, DIM), jnp.float32)]
        + [jax.ShapeDtypeStruct((N_BLKS, 1, ROW_BLK), jnp.int32)
           for _ in range(NQ)]
        + [
            jax.ShapeDtypeStruct((N_BLKS, NQ, K), jnp.float32),
            jax.ShapeDtypeStruct((N_BLKS, NQ, 128), jnp.float32),
        ],
        compiler_params=pltpu.CompilerParams(
            dimension_semantics=("parallel",)),
    )(x_flat, cb_flat, cbw, cbsq_all)

    quantized = outs[0]
    idx_list = outs[1:1 + NQ]
    counts_p, commit_p = outs[1 + NQ], outs[2 + NQ]

    _, com, perp = pl.pallas_call(
        _finalize_body,
        out_shape=[
            jax.ShapeDtypeStruct((NQ, K), jnp.float32),
            jax.ShapeDtypeStruct((8, 128), jnp.float32),
            jax.ShapeDtypeStruct((8, 128), jnp.float32),
        ],
    )(counts_p, commit_p)

    indices_out = jnp.stack(
        [ix.reshape(N) for ix in idx_list], axis=-1).reshape(B, S, NQ)
    quantized_out = quantized.reshape(B, S, DIM)
    return (quantized_out, indices_out, com[0, 0], perp[0, 0])


# speedup vs baseline: 1.1228x; 1.1228x over previous
import jax
import jax.numpy as jnp
from jax.experimental import pallas as pl
from jax.experimental.pallas import tpu as pltpu

B, S, DIM = 8, 576, 256
K = 1024
NQ = 8
N = B * S
COMMIT_W = 0.25

ROW_BLK = 768
N_BLKS = N // ROW_BLK


def _rvq_body(x_ref, cb_ref, cbw_ref, cbsq_ref, q_ref, *out_refs):
    idx_refs = out_refs[:NQ]
    counts_ref = out_refs[NQ]
    commit_ref = out_refs[NQ + 1]

    x = x_ref[...]
    iota = jax.lax.broadcasted_iota(jnp.int32, (ROW_BLK, K), 1)
    r = x
    qsum = jnp.zeros_like(x)
    rsq = jnp.sum(r ** 2, axis=1, keepdims=True)
    for level in range(NQ):
        cb = cb_ref[level * K:(level + 1) * K, :]
        cbsq = cbsq_ref[level:level + 1, :]
        sc = jax.lax.dot_general(
            r, cb, (((1,), (1,)), ((), ())),
            preferred_element_type=jnp.float32)
        d = (rsq - 2.0 * sc) + cbsq
        idx = jnp.argmin(d, axis=1).astype(jnp.int32)
        idx_refs[level][...] = idx[None, None, :]
        ohi = (iota == idx[:, None]).astype(jnp.int8)
        dn = (((1,), (0,)), ((), ()))
        sl = slice(level * K, (level + 1) * K)
        g = jax.lax.dot_general(ohi, cbw_ref[sl, :], dn,
                                preferred_element_type=jnp.int32)
        bits = (g[:, 0 * DIM:1 * DIM] + 128).astype(jnp.uint32)
        for c in range(1, 4):
            bits = bits | ((g[:, c * DIM:(c + 1) * DIM] + 128)
                           .astype(jnp.uint32) << (8 * c))
        q = jax.lax.bitcast_convert_type(bits, jnp.float32)
        counts_ref[0, level:level + 1, :] = jax.lax.dot_general(
            jnp.ones((1, ROW_BLK), jnp.int8), ohi,
            (((1,), (0,)), ((), ())),
            preferred_element_type=jnp.int32).astype(jnp.float32)
        qsum = qsum + q
        r = r - q
        rsq = jnp.sum(r ** 2, axis=1, keepdims=True)
        commit_ref[0, level:level + 1, :] = jnp.full((1, 128), jnp.sum(rsq))
    q_ref[...] = x + (qsum - x)


def _finalize_body(counts_ref, commit_ref, counts_out_ref, com_ref, perp_ref):
    counts = jnp.sum(counts_ref[...], axis=0)
    counts_out_ref[...] = counts
    p = counts / N
    ent = jnp.sum(p * jnp.log(p + 1e-10), axis=1, keepdims=True)
    perps = jnp.exp(-ent)
    perp_ref[...] = jnp.full((8, 128), jnp.sum(perps) / NQ)
    commit = jnp.sum(commit_ref[...][:, :, 0:1]) / (N * DIM)
    com_ref[...] = jnp.full((8, 128), commit * COMMIT_W)


def kernel(x, codebooks):
    x_flat = x.reshape(N, DIM)
    cb_flat = codebooks.reshape(NQ * K, DIM)
    cb_bits = jax.lax.bitcast_convert_type(cb_flat, jnp.uint32)
    cbw = jnp.concatenate(
        [((cb_bits >> (8 * c)) & 0xFF).astype(jnp.int32) - 128
         for c in range(4)],
        axis=1).astype(jnp.int8)
    cbsq_all = jnp.sum(codebooks ** 2, axis=-1)

    outs = pl.pallas_call(
        _rvq_body,
        grid=(N_BLKS,),
        in_specs=[
            pl.BlockSpec((ROW_BLK, DIM), lambda i: (i, 0)),
            pl.BlockSpec((NQ * K, DIM), lambda i: (0, 0)),
            pl.BlockSpec((NQ * K, 4 * DIM), lambda i: (0, 0)),
            pl.BlockSpec((NQ, K), lambda i: (0, 0)),
        ],
        out_specs=[pl.BlockSpec((ROW_BLK, DIM), lambda i: (i, 0))]
        + [pl.BlockSpec((1, 1, ROW_BLK), lambda i: (i, 0, 0))
           for _ in range(NQ)]
        + [
            pl.BlockSpec((1, NQ, K), lambda i: (i, 0, 0)),
            pl.BlockSpec((1, NQ, 128), lambda i: (i, 0, 0)),
        ],
        out_shape=[jax.ShapeDtypeStruct((N, DIM), jnp.float32)]
        + [jax.ShapeDtypeStruct((N_BLKS, 1, ROW_BLK), jnp.int32)
           for _ in range(NQ)]
        + [
            jax.ShapeDtypeStruct((N_BLKS, NQ, K), jnp.float32),
            jax.ShapeDtypeStruct((N_BLKS, NQ, 128), jnp.float32),
        ],
        compiler_params=pltpu.CompilerParams(
            dimension_semantics=("parallel",)),
    )(x_flat, cb_flat, cbw, cbsq_all)

    quantized = outs[0]
    idx_list = outs[1:1 + NQ]
    counts_p, commit_p = outs[1 + NQ], outs[2 + NQ]

    _, com, perp = pl.pallas_call(
        _finalize_body,
        out_shape=[
            jax.ShapeDtypeStruct((NQ, K), jnp.float32),
            jax.ShapeDtypeStruct((8, 128), jnp.float32),
            jax.ShapeDtypeStruct((8, 128), jnp.float32),
        ],
    )(counts_p, commit_p)

    indices_out = jnp.stack(
        [ix.reshape(N) for ix in idx_list], axis=-1).reshape(B, S, NQ)
    quantized_out = quantized.reshape(B, S, DIM)
    return (quantized_out, indices_out, com[0, 0], perp[0, 0])
